# baseline (speedup 1.0000x reference)
import functools

import numpy as np
import jax
import jax.numpy as jnp
from jax import lax
from jax.experimental import pallas as pl
from jax.experimental.pallas import tpu as pltpu
from jax.experimental.pallas import tpu_sc as plsc

N = 50000
E = 800000
HID = 64
F2 = 32
NB = 1024
NBLK = 49

B = 128
CE = 256
CN = 200
EROWS = 6528
EROWS2 = 3264
EPAD = EROWS * B
DCH = 51
NA = 50176
DUMMY = NA - 1
RPT = NA // 16
ZR = 56
OCH = 224

_f32 = jnp.float32


def _sc_agg_body(ht0, ht1, sd3, out0, out1, *scr):
    acc, zbuf = scr[0], scr[1]
    I = scr[2:5]
    g = scr[5:8]
    semI = scr[8:11]
    semG = scr[11:14]
    semS = scr[14:17]
    c = lax.axis_index("c")
    s = lax.axis_index("s")

    def _zb(i, car):
        zbuf[i, pl.ds(0, 16)] = jnp.zeros((16,), _f32)
        zbuf[i, pl.ds(16, 16)] = jnp.zeros((16,), _f32)
        return car
    lax.fori_loop(0, ZR, _zb, 0)

    def _zc(k, car):
        pltpu.sync_copy(zbuf, acc.at[pl.ds(s * RPT + k * ZR, ZR)])
        return car
    lax.fori_loop(0, RPT // ZR, _zc, 0)
    plsc.subcore_barrier()

    base = s * CN

    def _run(tab):
        def ifire(ch, q):
            pltpu.async_copy(sd3.at[pl.ds(base + ch, 2)], I[q], semI[q])

        def iwait(ch, q):
            pltpu.make_async_copy(sd3.at[pl.ds(base + ch, 2)], I[q],
                                  semI[q]).wait()

        def gfire(q, p, r):
            pltpu.async_copy(tab.at[I[q].at[p, 0]], g[r], semG[r])

        def gwait(q, p, r):
            pltpu.make_async_copy(tab.at[I[q].at[p, 0]], g[r],
                                  semG[r]).wait()

        def sfire(q, p, r):
            pltpu.async_copy(g[r], acc.at[I[q].at[p, 1]], semS[r], add=True)

        def swait(q, p, r):
            pltpu.make_async_copy(g[r], acc.at[I[q].at[p, 1]],
                                  semS[r]).wait()

        def slot(ch, j, first, pro=False):
            p, q, r = j % 2, (j // 2) % 3, j % 3
            j2 = (j - 2) % 6
            jn = (j + 1) % 6
            if not first:
                swait((j2 // 2) % 3, j2 % 2, j2 % 3)
            if p == 1:
                iwait(ch + 1, (jn // 2) % 3)
            gfire((jn // 2) % 3, jn % 2, jn % 3)
            gwait(q, p, r)
            sfire(q, p, r)
            if p == 1 and not (pro and ch < 3):
                j3 = (j + 3) % 6
                ifire(ch + 3, (j3 // 2) % 3)
        for q in range(3):
            ifire(2 * q, q)
        iwait(0, 0)
        gfire(0, 0, 0)
        for ch in range(8):
            slot(ch, ch % 6, ch < 2, pro=True)

        def _body(k, car):
            c0 = 8 + 6 * k
            for i in range(6):
                slot(c0 + i, (2 + i) % 6, False)
            return car
        lax.fori_loop(0, (CN - 8) // 6, _body, 0)

        swait(0, 0, 0)
        swait(0, 1, 1)
        gwait(1, 0, 2)
        iwait(CN + 2, 2)

    @pl.when(c == 0)
    def _():
        _run(ht0)

    @pl.when(c == 1)
    def _():
        _run(ht1)

    plsc.subcore_barrier()

    def _out(dst):
        def _cp(k, car):
            r = s * RPT + k * OCH
            pltpu.sync_copy(acc.at[pl.ds(r, OCH)], g[0].at[pl.ds(0, OCH)])
            pltpu.sync_copy(g[0].at[pl.ds(0, OCH)], dst.at[pl.ds(r, OCH)])
            return car
        lax.fori_loop(0, RPT // OCH, _cp, 0)

    @pl.when(c == 0)
    def _():
        _out(out0)

    @pl.when(c == 1)
    def _():
        _out(out1)


def _sc_deg_body(sd3, outd, sd0, accd, zbufd, sdbuf, ones, sem):
    c = lax.axis_index("c")
    s = lax.axis_index("s")

    def _zb(i, car):
        zbufd[pl.ds(i * 16, 16)] = jnp.zeros((16,), _f32)
        return car
    lax.fori_loop(0, RPT // 16, _zb, 0)
    for t in range(16):
        ones[pl.ds(t * 16, 16)] = jnp.ones((16,), _f32)
    pltpu.sync_copy(zbufd, accd.at[pl.ds(s * RPT, RPT)])
    plsc.subcore_barrier()

    rbase = c * (EROWS2 // 2) + s * (2 * DCH)

    def _chunk(j, car):
        r0 = rbase + j * 2
        pltpu.sync_copy(sd3.at[pl.ds(r0, 2)], sdbuf)
        for k in range(2):
            pltpu.async_copy(ones, accd.at[sdbuf.at[k, 1]], sem,
                             add=True)
        for k in range(2):
            pltpu.make_async_copy(ones, accd.at[sdbuf.at[k, 1]],
                                  sem).wait()
        for k in range(2):
            def _cmp(i, car2):
                sv = sdbuf[k, 0, pl.ds(i * 16, 16)]
                dv = sdbuf[k, 1, pl.ds(i * 16, 16)]
                sdbuf[k, 1, pl.ds(i * 16, 16)] = jnp.where(
                    sv == dv, jnp.full((16,), DUMMY, jnp.int32), dv)
                return car2
            lax.fori_loop(0, CE // 16, _cmp, 0)
        pltpu.sync_copy(sdbuf, sd0.at[pl.ds(r0, 2)])
        return car
    lax.fori_loop(0, DCH, _chunk, 0)
    plsc.subcore_barrier()
    pltpu.sync_copy(accd.at[pl.ds(s * RPT, RPT)], zbufd)
    pltpu.sync_copy(zbufd, outd.at[pl.ds(c * NA + s * RPT, RPT)])


_SC_MESH = plsc.VectorSubcoreMesh(core_axis_name="c", subcore_axis_name="s")

_sc_agg = pl.kernel(
    _sc_agg_body,
    out_type=[jax.ShapeDtypeStruct((NA, F2), _f32),
              jax.ShapeDtypeStruct((NA, F2), _f32)],
    mesh=_SC_MESH,
    compiler_params=pltpu.CompilerParams(use_tc_tiling_on_sc=False),
    scratch_types=(
        [pltpu.VMEM_SHARED((NA, F2), _f32), pltpu.VMEM((ZR, F2), _f32)]
        + [pltpu.VMEM((2, 2, 2 * B), jnp.int32)] * 3
        + [pltpu.VMEM((CE, F2), _f32)] * 3
        + [pltpu.SemaphoreType.DMA] * 9
    ),
)

_sc_deg = pl.kernel(
    _sc_deg_body,
    out_type=[jax.ShapeDtypeStruct((2 * NA,), _f32),
              jax.ShapeDtypeStruct((EROWS2, 2, 2 * B), jnp.int32)],
    mesh=_SC_MESH,
    compiler_params=pltpu.CompilerParams(use_tc_tiling_on_sc=False),
    scratch_types=[
        pltpu.VMEM_SHARED((NA,), _f32),
        pltpu.VMEM((RPT,), _f32),
        pltpu.VMEM((2, 2, CE), jnp.int32),
        pltpu.VMEM((CE,), _f32),
        pltpu.SemaphoreType.DMA,
    ],
)


def _dot(a, b):
    return jnp.dot(a, b, preferred_element_type=_f32)


def _dotT(a, b):
    return lax.dot_general(a, b, (((0,), (0,)), ((), ())),
                           preferred_element_type=_f32)


def _tc_a_body(xT, bdT, M, bc, wsy, wsb, bs, wny, wnb, bn, sx, h0, h1):
    ry = jax.nn.relu(_dotT(xT[...], M[...]) + bc[...])
    rbT = jax.nn.relu(bdT[...])
    sx[...] = _dot(ry, wsy[...]) + _dotT(rbT, wsb[...]) + bs[...]
    xnb = _dot(ry, wny[...]) + _dotT(rbT, wnb[...]) + bn[...]
    h0[...] = xnb[:, :F2]
    h1[...] = xnb[:, F2:]


def _tc_1_body(sx, a0, a1, d0, d1, wrt, bl, h0, h1, hr, dinv):
    h = sx[...] + jnp.concatenate([a0[...], a1[...]], axis=1)
    h0[...] = h[:, :F2]
    h1[...] = h[:, F2:]
    hr[...] = _dot(h, wrt[...]) + bl[...]
    dinv[...] = 1.0 / jnp.maximum(d0[...] + d1[...], 1.0)


def _tc_mid_body(a0, a1, dinv, hri, wlt, g, b, wrt, bl, h0, h1, hro):
    dv = dinv[...].reshape(NB, 1)
    u = _dot(jnp.concatenate([a0[...], a1[...]], axis=1) * dv,
             wlt[...]) + hri[...]
    v = jax.nn.relu(u)
    mu = jnp.mean(v, axis=1, keepdims=True)
    var = jnp.mean((v - mu) ** 2, axis=1, keepdims=True)
    h = (v - mu) * lax.rsqrt(var + 1e-5) * g[...] + b[...]
    h0[...] = h[:, :F2]
    h1[...] = h[:, F2:]
    hro[...] = _dot(h, wrt[...]) + bl[...]


def _tc_4_body(a0, a1, dinv, hri, wlt, w1, b1, w2, b2, emb, lsm):
    dv = dinv[...].reshape(NB, 1)
    e = _dot(jnp.concatenate([a0[...], a1[...]], axis=1) * dv,
             wlt[...]) + hri[...]
    emb[...] = e
    p = _dot(jax.nn.relu(e), w1[...]) + b1[...]
    q = _dot(p, w2[...]) + b2[...]
    m = jnp.max(q, axis=1, keepdims=True)
    lsm[...] = q - (jnp.log(jnp.sum(jnp.exp(q - m), axis=1, keepdims=True))
                    + m)


def _rows(shape):
    return pl.BlockSpec((NB,) + shape[1:], lambda i: (i,) + (0,) * (len(shape) - 1))


def _full(shape):
    return pl.BlockSpec(shape, lambda i: (0,) * len(shape))


def _spec(shape, sel=0):
    if shape[0] in (N, NA):
        return _rows(shape)
    if len(shape) == 3 and shape[-1] in (N, NA):
        return pl.BlockSpec((1, 1, NB), lambda i, s=sel: (s, 0, i))
    if len(shape) == 2 and shape[-1] == N:
        return pl.BlockSpec((shape[0], NB), lambda i: (0, i))
    return _full(shape)


def _tc_call(body, ins, n_out, out_shapes, sels=None):
    specs = [_spec(a.shape, 0 if sels is None else sels[k])
             for k, a in enumerate(ins)]
    return pl.pallas_call(
        body,
        grid=(NBLK,),
        in_specs=specs,
        out_specs=[_spec(s) for s in out_shapes],
        out_shape=[jax.ShapeDtypeStruct(s, _f32) for s in out_shapes],
    )(*ins)


def _build_conv_matrix(Wc, bc):
    o, c, di, dj, p, q = np.meshgrid(np.arange(3), np.arange(3), np.arange(3),
                                     np.arange(3), np.arange(6), np.arange(6),
                                     indexing="ij")
    k = (c * 64 + (p + di) * 8 + (q + dj)).ravel()
    m = (o * 36 + p * 6 + q).ravel()
    w = Wc[o.ravel(), c.ravel(), di.ravel(), dj.ravel()]
    M = jnp.zeros((192, 108), _f32).at[k, m].add(w)
    return M, jnp.repeat(bc, 36)[None, :]


def kernel(x, bd_pred, Wc, bc, Wlin, blin, Wlins, blins, Wl1, bl1, Wr1, Wl2,
           bl2, Wr2, Wl3, bl3, Wr3, ln1_g, ln1_b, ln2_g, ln2_b, mp1_W, mp1_b,
           mp2_W, mp2_b, edge_index):
    M, bcvec = _build_conv_matrix(Wc, bc)
    xT = x.reshape(N, 192).T
    bdT = bd_pred.T
    pad = EPAD - E
    src2 = jnp.concatenate(
        [edge_index[0], jnp.zeros((pad,), jnp.int32)]).reshape(EROWS2, 1, 2 * B)
    dst2 = jnp.concatenate(
        [edge_index[1],
         jnp.full((pad,), DUMMY, jnp.int32)]).reshape(EROWS2, 1, 2 * B)
    sd2 = jnp.concatenate([src2, dst2], axis=1)

    row = lambda v: v[None, :]

    outd, sd0 = _sc_deg(sd2)
    outd3 = outd.reshape(2, 1, NA)[:, :, :N] if False else outd.reshape(2, 1, NA)

    sx, h0, h1 = _tc_call(
        _tc_a_body,
        [xT, bdT, M, bcvec, Wlins[:, :108].T, Wlins[:, 108:].T, row(blins),
         Wlin[:, :108].T, Wlin[:, 108:].T, row(blin)],
        3, [(N, HID), (N, F2), (N, F2)])

    a0, a1 = _sc_agg(h0, h1, sd0)
    h0, h1, hr, dinv = _tc_call(
        _tc_1_body,
        [sx, a0, a1, outd3, outd3, Wr1.T, row(bl1)],
        4, [(N, F2), (N, F2), (N, HID), (1, 1, N)],
        sels=[0, 0, 0, 0, 1, 0, 0])

    a0, a1 = _sc_agg(h0, h1, sd2)
    h0, h1, hr = _tc_call(
        _tc_mid_body,
        [a0, a1, dinv, hr, Wl1.T, row(ln1_g), row(ln1_b), Wr2.T, row(bl2)],
        3, [(N, F2), (N, F2), (N, HID)])

    a0, a1 = _sc_agg(h0, h1, sd2)
    h0, h1, hr = _tc_call(
        _tc_mid_body,
        [a0, a1, dinv, hr, Wl2.T, row(ln2_g), row(ln2_b), Wr3.T, row(bl3)],
        3, [(N, F2), (N, F2), (N, HID)])

    a0, a1 = _sc_agg(h0, h1, sd2)
    emb, lsm = _tc_call(
        _tc_4_body,
        [a0, a1, dinv, hr, Wl3.T, mp1_W.T, row(mp1_b), mp2_W.T, row(mp2_b)],
        2, [(N, HID), (N, 8)])

    return emb, lsm


# speedup vs baseline: 1.0052x; 1.0052x over previous
import functools

import numpy as np
import jax
import jax.numpy as jnp
from jax import lax
from jax.experimental import pallas as pl
from jax.experimental.pallas import tpu as pltpu
from jax.experimental.pallas import tpu_sc as plsc

N = 50000
E = 800000
HID = 64
F2 = 32
NB = 1024
NBLK = 49

B = 128
CE = 256
CN = 200
EROWS = 6528
EROWS2 = 3264
EPAD = EROWS * B
DCH = 51
NA = 50176
DUMMY = NA - 1
RPT = NA // 16
ZR = 56
OCH = 224

_f32 = jnp.float32


def _sc_agg_body(ht0, ht1, sd3, out0, out1, *scr):
    acc, zbuf = scr[0], scr[1]
    I = scr[2:8]
    g = scr[8:11]
    semI = scr[11:17]
    semG = scr[17:20]
    semS = scr[20:23]
    c = lax.axis_index("c")
    s = lax.axis_index("s")

    def _zb(i, car):
        zbuf[i, pl.ds(0, 16)] = jnp.zeros((16,), _f32)
        zbuf[i, pl.ds(16, 16)] = jnp.zeros((16,), _f32)
        return car
    lax.fori_loop(0, ZR, _zb, 0)

    def _zc(k, car):
        pltpu.sync_copy(zbuf, acc.at[pl.ds(s * RPT + k * ZR, ZR)])
        return car
    lax.fori_loop(0, RPT // ZR, _zc, 0)
    plsc.subcore_barrier()

    base = s * CN

    def _run(tab):
        def ifire(ch, q):
            pltpu.async_copy(sd3.at[base + ch], I[q], semI[q])

        def iwait(ch, q):
            pltpu.make_async_copy(sd3.at[base + ch], I[q], semI[q]).wait()

        def gfire(q):
            pltpu.async_copy(tab.at[I[q].at[0]], g[q % 3], semG[q % 3])

        def gwait(q):
            pltpu.make_async_copy(tab.at[I[q].at[0]], g[q % 3],
                                  semG[q % 3]).wait()

        def sfire(q):
            pltpu.async_copy(g[q % 3], acc.at[I[q].at[1]], semS[q % 3],
                             add=True)

        def swait(q):
            pltpu.make_async_copy(g[q % 3], acc.at[I[q].at[1]],
                                  semS[q % 3]).wait()

        def slot(ch, j, first):
            if not first:
                swait((j - 2) % 6)
            iwait(ch + 1, (j + 1) % 6)
            gfire((j + 1) % 6)
            gwait(j)
            sfire(j)
            ifire(ch + 4, (j + 4) % 6)

        for q in range(4):
            ifire(q, q)
        iwait(0, 0)
        gfire(0)
        for ch in range(8):
            slot(ch, ch % 6, ch < 2)

        def _body(k, car):
            c0 = 8 + 6 * k
            for i in range(6):
                slot(c0 + i, (2 + i) % 6, False)
            return car
        lax.fori_loop(0, (CN - 8) // 6, _body, 0)

        swait(0)
        swait(1)
        gwait(2)
        for i in range(3):
            iwait(CN + 1 + i, 3 + i)

    @pl.when(c == 0)
    def _():
        _run(ht0)

    @pl.when(c == 1)
    def _():
        _run(ht1)

    plsc.subcore_barrier()

    def _out(dst):
        def _cp(k, car):
            r = s * RPT + k * OCH
            pltpu.sync_copy(acc.at[pl.ds(r, OCH)], g[0].at[pl.ds(0, OCH)])
            pltpu.sync_copy(g[0].at[pl.ds(0, OCH)], dst.at[pl.ds(r, OCH)])
            return car
        lax.fori_loop(0, RPT // OCH, _cp, 0)

    @pl.when(c == 0)
    def _():
        _out(out0)

    @pl.when(c == 1)
    def _():
        _out(out1)


def _sc_deg_body(sd3, outd, sd0, accd, zbufd, sdbuf, ones, sem):
    c = lax.axis_index("c")
    s = lax.axis_index("s")

    def _zb(i, car):
        zbufd[pl.ds(i * 16, 16)] = jnp.zeros((16,), _f32)
        return car
    lax.fori_loop(0, RPT // 16, _zb, 0)
    for t in range(16):
        ones[pl.ds(t * 16, 16)] = jnp.ones((16,), _f32)
    pltpu.sync_copy(zbufd, accd.at[pl.ds(s * RPT, RPT)])
    plsc.subcore_barrier()

    rbase = c * (EROWS2 // 2) + s * (2 * DCH)

    def _chunk(j, car):
        r0 = rbase + j * 2
        pltpu.sync_copy(sd3.at[pl.ds(r0, 2)], sdbuf)
        for k in range(2):
            pltpu.async_copy(ones, accd.at[sdbuf.at[k, 1]], sem,
                             add=True)
        for k in range(2):
            pltpu.make_async_copy(ones, accd.at[sdbuf.at[k, 1]],
                                  sem).wait()
        for k in range(2):
            def _cmp(i, car2):
                sv = sdbuf[k, 0, pl.ds(i * 16, 16)]
                dv = sdbuf[k, 1, pl.ds(i * 16, 16)]
                sdbuf[k, 1, pl.ds(i * 16, 16)] = jnp.where(
                    sv == dv, jnp.full((16,), DUMMY, jnp.int32), dv)
                return car2
            lax.fori_loop(0, CE // 16, _cmp, 0)
        pltpu.sync_copy(sdbuf, sd0.at[pl.ds(r0, 2)])
        return car
    lax.fori_loop(0, DCH, _chunk, 0)
    plsc.subcore_barrier()
    pltpu.sync_copy(accd.at[pl.ds(s * RPT, RPT)], zbufd)
    pltpu.sync_copy(zbufd, outd.at[pl.ds(c * NA + s * RPT, RPT)])


_SC_MESH = plsc.VectorSubcoreMesh(core_axis_name="c", subcore_axis_name="s")

_sc_agg = pl.kernel(
    _sc_agg_body,
    out_type=[jax.ShapeDtypeStruct((NA, F2), _f32),
              jax.ShapeDtypeStruct((NA, F2), _f32)],
    mesh=_SC_MESH,
    compiler_params=pltpu.CompilerParams(use_tc_tiling_on_sc=False),
    scratch_types=(
        [pltpu.VMEM_SHARED((NA, F2), _f32), pltpu.VMEM((ZR, F2), _f32)]
        + [pltpu.VMEM((2, 2 * B), jnp.int32)] * 6
        + [pltpu.VMEM((CE, F2), _f32)] * 3
        + [pltpu.SemaphoreType.DMA] * 12
    ),
)

_sc_deg = pl.kernel(
    _sc_deg_body,
    out_type=[jax.ShapeDtypeStruct((2 * NA,), _f32),
              jax.ShapeDtypeStruct((EROWS2, 2, 2 * B), jnp.int32)],
    mesh=_SC_MESH,
    compiler_params=pltpu.CompilerParams(use_tc_tiling_on_sc=False),
    scratch_types=[
        pltpu.VMEM_SHARED((NA,), _f32),
        pltpu.VMEM((RPT,), _f32),
        pltpu.VMEM((2, 2, CE), jnp.int32),
        pltpu.VMEM((CE,), _f32),
        pltpu.SemaphoreType.DMA,
    ],
)


def _dot(a, b):
    return jnp.dot(a, b, preferred_element_type=_f32)


def _dotT(a, b):
    return lax.dot_general(a, b, (((0,), (0,)), ((), ())),
                           preferred_element_type=_f32)


def _tc_a_body(xT, bdT, M, bc, wsy, wsb, bs, wny, wnb, bn, sx, h0, h1):
    ry = jax.nn.relu(_dotT(xT[...], M[...]) + bc[...])
    rbT = jax.nn.relu(bdT[...])
    sx[...] = _dot(ry, wsy[...]) + _dotT(rbT, wsb[...]) + bs[...]
    xnb = _dot(ry, wny[...]) + _dotT(rbT, wnb[...]) + bn[...]
    h0[...] = xnb[:, :F2]
    h1[...] = xnb[:, F2:]


def _tc_1_body(sx, a0, a1, d0, d1, wrt, bl, h0, h1, hr, dinv):
    h = sx[...] + jnp.concatenate([a0[...], a1[...]], axis=1)
    h0[...] = h[:, :F2]
    h1[...] = h[:, F2:]
    hr[...] = _dot(h, wrt[...]) + bl[...]
    dinv[...] = 1.0 / jnp.maximum(d0[...] + d1[...], 1.0)


def _tc_mid_body(a0, a1, dinv, hri, wlt, g, b, wrt, bl, h0, h1, hro):
    dv = dinv[...].reshape(NB, 1)
    u = _dot(jnp.concatenate([a0[...], a1[...]], axis=1) * dv,
             wlt[...]) + hri[...]
    v = jax.nn.relu(u)
    mu = jnp.mean(v, axis=1, keepdims=True)
    var = jnp.mean((v - mu) ** 2, axis=1, keepdims=True)
    h = (v - mu) * lax.rsqrt(var + 1e-5) * g[...] + b[...]
    h0[...] = h[:, :F2]
    h1[...] = h[:, F2:]
    hro[...] = _dot(h, wrt[...]) + bl[...]


def _tc_4_body(a0, a1, dinv, hri, wlt, w1, b1, w2, b2, emb, lsm):
    dv = dinv[...].reshape(NB, 1)
    e = _dot(jnp.concatenate([a0[...], a1[...]], axis=1) * dv,
             wlt[...]) + hri[...]
    emb[...] = e
    p = _dot(jax.nn.relu(e), w1[...]) + b1[...]
    q = _dot(p, w2[...]) + b2[...]
    m = jnp.max(q, axis=1, keepdims=True)
    lsm[...] = q - (jnp.log(jnp.sum(jnp.exp(q - m), axis=1, keepdims=True))
                    + m)


def _rows(shape):
    return pl.BlockSpec((NB,) + shape[1:], lambda i: (i,) + (0,) * (len(shape) - 1))


def _full(shape):
    return pl.BlockSpec(shape, lambda i: (0,) * len(shape))


def _spec(shape, sel=0):
    if shape[0] in (N, NA):
        return _rows(shape)
    if len(shape) == 3 and shape[-1] in (N, NA):
        return pl.BlockSpec((1, 1, NB), lambda i, s=sel: (s, 0, i))
    if len(shape) == 2 and shape[-1] == N:
        return pl.BlockSpec((shape[0], NB), lambda i: (0, i))
    return _full(shape)


def _tc_call(body, ins, n_out, out_shapes, sels=None):
    specs = [_spec(a.shape, 0 if sels is None else sels[k])
             for k, a in enumerate(ins)]
    return pl.pallas_call(
        body,
        grid=(NBLK,),
        in_specs=specs,
        out_specs=[_spec(s) for s in out_shapes],
        out_shape=[jax.ShapeDtypeStruct(s, _f32) for s in out_shapes],
    )(*ins)


def _build_conv_matrix(Wc, bc):
    o, c, di, dj, p, q = np.meshgrid(np.arange(3), np.arange(3), np.arange(3),
                                     np.arange(3), np.arange(6), np.arange(6),
                                     indexing="ij")
    k = (c * 64 + (p + di) * 8 + (q + dj)).ravel()
    m = (o * 36 + p * 6 + q).ravel()
    w = Wc[o.ravel(), c.ravel(), di.ravel(), dj.ravel()]
    M = jnp.zeros((192, 108), _f32).at[k, m].add(w)
    return M, jnp.repeat(bc, 36)[None, :]


def kernel(x, bd_pred, Wc, bc, Wlin, blin, Wlins, blins, Wl1, bl1, Wr1, Wl2,
           bl2, Wr2, Wl3, bl3, Wr3, ln1_g, ln1_b, ln2_g, ln2_b, mp1_W, mp1_b,
           mp2_W, mp2_b, edge_index):
    M, bcvec = _build_conv_matrix(Wc, bc)
    xT = x.reshape(N, 192).T
    bdT = bd_pred.T
    pad = EPAD - E
    src2 = jnp.concatenate(
        [edge_index[0], jnp.zeros((pad,), jnp.int32)]).reshape(EROWS2, 1, 2 * B)
    dst2 = jnp.concatenate(
        [edge_index[1],
         jnp.full((pad,), DUMMY, jnp.int32)]).reshape(EROWS2, 1, 2 * B)
    sd2 = jnp.concatenate([src2, dst2], axis=1)

    row = lambda v: v[None, :]

    outd, sd0 = _sc_deg(sd2)
    outd3 = outd.reshape(2, 1, NA)[:, :, :N] if False else outd.reshape(2, 1, NA)

    sx, h0, h1 = _tc_call(
        _tc_a_body,
        [xT, bdT, M, bcvec, Wlins[:, :108].T, Wlins[:, 108:].T, row(blins),
         Wlin[:, :108].T, Wlin[:, 108:].T, row(blin)],
        3, [(N, HID), (N, F2), (N, F2)])

    a0, a1 = _sc_agg(h0, h1, sd0)
    h0, h1, hr, dinv = _tc_call(
        _tc_1_body,
        [sx, a0, a1, outd3, outd3, Wr1.T, row(bl1)],
        4, [(N, F2), (N, F2), (N, HID), (1, 1, N)],
        sels=[0, 0, 0, 0, 1, 0, 0])

    a0, a1 = _sc_agg(h0, h1, sd2)
    h0, h1, hr = _tc_call(
        _tc_mid_body,
        [a0, a1, dinv, hr, Wl1.T, row(ln1_g), row(ln1_b), Wr2.T, row(bl2)],
        3, [(N, F2), (N, F2), (N, HID)])

    a0, a1 = _sc_agg(h0, h1, sd2)
    h0, h1, hr = _tc_call(
        _tc_mid_body,
        [a0, a1, dinv, hr, Wl2.T, row(ln2_g), row(ln2_b), Wr3.T, row(bl3)],
        3, [(N, F2), (N, F2), (N, HID)])

    a0, a1 = _sc_agg(h0, h1, sd2)
    emb, lsm = _tc_call(
        _tc_4_body,
        [a0, a1, dinv, hr, Wl3.T, mp1_W.T, row(mp1_b), mp2_W.T, row(mp2_b)],
        2, [(N, HID), (N, 8)])

    return emb, lsm
